# baseline (speedup 1.0000x reference)
import functools

import jax
import jax.numpy as jnp
from jax import lax
from jax.experimental import pallas as pl
from jax.experimental.pallas import tpu as pltpu
from jax.experimental.pallas import tpu_sc as plsc

NUM_FIELDS = 26
CARD1 = 100001
D = 32
BATCH = 16384
NC, NS, L = 2, 16, 16
NW = NC * NS
ROWS = BATCH * NUM_FIELDS
RPW = ROWS // NW
IDX_MINOR = 128
IDX_MAJOR = RPW // IDX_MINOR
CHUNK = 512
G_PER_CHUNK = CHUNK // IDX_MINOR
N_CHUNKS = RPW // CHUNK

_mesh = plsc.VectorSubcoreMesh(
    core_axis_name="c", subcore_axis_name="s", num_cores=NC, num_subcores=NS
)


@functools.partial(
    pl.kernel,
    out_type=jax.ShapeDtypeStruct((ROWS, D), jnp.float32),
    mesh=_mesh,
    scratch_types=[
        pltpu.VMEM((IDX_MAJOR, IDX_MINOR), jnp.int32),
        pltpu.VMEM((CHUNK, D), jnp.float32),
        pltpu.SemaphoreType.DMA,
    ],
    compiler_params=pltpu.CompilerParams(use_tc_tiling_on_sc=False),
)
def _emb_gather(x_hbm, table_hbm, out_hbm, idx_v, buf_v, sem):
    wid = lax.axis_index("s") * NC + lax.axis_index("c")
    pltpu.sync_copy(x_hbm.at[wid], idx_v)

    def chunk_body(c, carry):
        for jj in range(G_PER_CHUNK):
            j = c * G_PER_CHUNK + jj
            for k in range(IDX_MINOR // L):
                p = c * CHUNK + jj * IDX_MINOR + k * L
                lanes = lax.iota(jnp.int32, L) + p
                field = lax.rem(lanes, NUM_FIELDS)
                idx_v[j, pl.ds(k * L, L)] = (
                    idx_v[j, pl.ds(k * L, L)] + field * CARD1
                )
        descs = []
        for jj in range(G_PER_CHUNK):
            j = c * G_PER_CHUNK + jj
            descs.append(
                pltpu.async_copy(
                    table_hbm.at[idx_v.at[j]],
                    buf_v.at[pl.ds(jj * IDX_MINOR, IDX_MINOR)],
                    sem,
                )
            )
        for d_ in descs:
            d_.wait()
        row0 = wid * RPW + c * CHUNK
        pltpu.sync_copy(buf_v, out_hbm.at[pl.ds(row0, CHUNK)])
        return carry

    lax.fori_loop(0, N_CHUNKS, chunk_body, 0)


def kernel(x, tables):
    xr = x.reshape(NW, IDX_MAJOR, IDX_MINOR)
    tf = tables.reshape(NUM_FIELDS * CARD1, D)
    out = _emb_gather(xr, tf)
    return out.reshape(BATCH, NUM_FIELDS, D)


# speedup vs baseline: 2.4388x; 2.4388x over previous
import functools

import jax
import jax.numpy as jnp
from jax import lax
from jax.experimental import pallas as pl
from jax.experimental.pallas import tpu as pltpu
from jax.experimental.pallas import tpu_sc as plsc

NUM_FIELDS = 26
CARD1 = 100001
D = 32
BATCH = 16384
NC, NS, L = 2, 16, 16
NW = NC * NS
BPW = BATCH // NW
IDX_MINOR = 128
G_PER_FIELD = BPW // IDX_MINOR

_mesh = plsc.VectorSubcoreMesh(
    core_axis_name="c", subcore_axis_name="s", num_cores=NC, num_subcores=NS
)


@functools.partial(
    pl.kernel,
    out_type=jax.ShapeDtypeStruct((BATCH, NUM_FIELDS, D), jnp.float32),
    mesh=_mesh,
    scratch_types=[
        pltpu.VMEM((NUM_FIELDS, G_PER_FIELD, IDX_MINOR), jnp.int32),
        pltpu.VMEM((BPW, D), jnp.float32),
        pltpu.SemaphoreType.DMA,
    ],
    compiler_params=pltpu.CompilerParams(use_tc_tiling_on_sc=False),
)
def _emb_gather(xw_hbm, tables_hbm, out_hbm, idx_v, buf_v, sem):
    wid = lax.axis_index("s") * NC + lax.axis_index("c")
    b0 = wid * BPW
    pltpu.sync_copy(xw_hbm.at[wid], idx_v)

    def field_body(f, carry):
        descs = []
        for k in range(G_PER_FIELD):
            descs.append(
                pltpu.async_copy(
                    tables_hbm.at[f].at[idx_v.at[f, k]],
                    buf_v.at[pl.ds(k * IDX_MINOR, IDX_MINOR)],
                    sem,
                )
            )
        for d_ in descs:
            d_.wait()
        pltpu.sync_copy(buf_v, out_hbm.at[pl.ds(b0, BPW), f])
        return carry

    lax.fori_loop(0, NUM_FIELDS, field_body, 0)


def kernel(x, tables):
    xw = (
        x.reshape(NW, BPW, NUM_FIELDS)
        .transpose(0, 2, 1)
        .reshape(NW, NUM_FIELDS, G_PER_FIELD, IDX_MINOR)
    )
    return _emb_gather(xw, tables)
